# Initial kernel scaffold; baseline (speedup 1.0000x reference)
import jax
import jax.numpy as jnp
from jax.experimental import pallas as pl


def kernel(x, edge_index, W1, b1, W2, b2):
    raise NotImplementedError("write your pallas kernel here")


# speedup vs baseline: 4.8568x; 4.8568x over previous
import functools

import jax
import jax.numpy as jnp
from jax import lax
from jax.experimental import pallas as pl
from jax.experimental.pallas import tpu as pltpu
from jax.experimental.pallas import tpu_sc as plsc

N_NODES = 10000
N_EDGES = 160000
D_FEAT = 256
EPSILON = 0.1
C = 10.0

NC = 2
NS = 16
DH = D_FEAT // NC
E_TILE = N_EDGES // NS
K = 80
CH = E_TILE // K
RCH = 80
NRC = N_NODES // RCH
NRC_TILE = (NRC + NS - 1) // NS


def _activation(x):
    mask = (x > EPSILON).astype(x.dtype)
    theta = (x - EPSILON) / (1.0 - EPSILON + 1e-8)
    theta = jnp.clip(theta, 0.0, 1.0)
    numerator = 1.0 + jnp.exp(jnp.asarray(-C, dtype=x.dtype))
    denominator = 1.0 + jnp.exp(-C * theta)
    return mask * (theta * numerator / denominator)


def _mm_body(x_ref, w_ref, o_ref):
    s = jnp.dot(x_ref[...], w_ref[...], preferred_element_type=jnp.float32)
    o_ref[0] = s[:, :DH]
    o_ref[1] = s[:, DH:]


def _tc_matmul(x, w):
    return pl.pallas_call(
        _mm_body,
        out_shape=jax.ShapeDtypeStruct((NC, N_NODES, DH), jnp.float32),
    )(x, w)


def _act_mm_body(a_ref, b_ref, w_ref, o_ref):
    a = jnp.concatenate([a_ref[0], a_ref[1]], axis=1) + b_ref[...]
    h = _activation(a)
    s = jnp.dot(h, w_ref[...], preferred_element_type=jnp.float32)
    o_ref[0] = s[:, :DH]
    o_ref[1] = s[:, DH:]


def _tc_act_matmul(agg, b, w):
    return pl.pallas_call(
        _act_mm_body,
        out_shape=jax.ShapeDtypeStruct((NC, N_NODES, DH), jnp.float32),
    )(agg, b, w)


def _bias_body(a_ref, b_ref, o_ref):
    o_ref[...] = jnp.concatenate([a_ref[0], a_ref[1]], axis=1) + b_ref[...]


def _tc_bias(agg, b):
    return pl.pallas_call(
        _bias_body,
        out_shape=jax.ShapeDtypeStruct((N_NODES, D_FEAT), jnp.float32),
    )(agg, b)


def _agg_body(support_ref, srcz_ref, dst_ref, zeros_ref, out_ref,
              src_v, dst_v, rows_v, acc, sem):
    c = lax.axis_index("c")
    s = lax.axis_index("s")

    def zero_body(i, carry):
        j = s * NRC_TILE + i

        @pl.when(j < NRC)
        def _():
            pltpu.sync_copy(zeros_ref, acc.at[pl.ds(j * RCH, RCH)])

        return carry

    lax.fori_loop(0, NRC_TILE, zero_body, 0)
    pltpu.sync_copy(srcz_ref.at[c, s], src_v)
    pltpu.sync_copy(dst_ref.at[s], dst_v)
    plsc.subcore_barrier()

    def body(j, carry):
        pltpu.async_copy(support_ref.at[src_v.at[j]], rows_v, sem).wait()
        pltpu.sync_copy(rows_v, acc.at[dst_v.at[j]], add=True)
        return carry

    lax.fori_loop(0, CH, body, 0)
    plsc.subcore_barrier()

    def out_body(i, carry):
        j = s * NRC_TILE + i

        @pl.when(j < NRC)
        def _():
            pltpu.sync_copy(acc.at[pl.ds(j * RCH, RCH)],
                            out_ref.at[c, pl.ds(j * RCH, RCH)])

        return carry

    lax.fori_loop(0, NRC_TILE, out_body, 0)


_agg_kernel = functools.partial(
    pl.kernel,
    out_type=jax.ShapeDtypeStruct((NC, N_NODES, DH), jnp.float32),
    mesh=plsc.VectorSubcoreMesh(core_axis_name="c", subcore_axis_name="s"),
    scratch_types=[
        pltpu.VMEM((CH, K), jnp.int32),
        pltpu.VMEM((CH, K), jnp.int32),
        pltpu.VMEM((K, DH), jnp.float32),
        pltpu.VMEM_SHARED((N_NODES, DH), jnp.float32),
        pltpu.SemaphoreType.DMA,
    ],
)(_agg_body)


def _sc_aggregate(support, srcz, dstr, zeros):
    return _agg_kernel(support, srcz, dstr, zeros)


def kernel(x, edge_index, W1, b1, W2, b2):
    src = edge_index[0].astype(jnp.int32)
    dst = edge_index[1].astype(jnp.int32)
    srcz = (src.reshape(1, NS, CH, K)
            + (jnp.arange(NC, dtype=jnp.int32) * N_NODES).reshape(NC, 1, 1, 1))
    dstr = dst.reshape(NS, CH, K)
    zeros = jnp.zeros((RCH, DH), jnp.float32)

    s1 = _tc_matmul(x, W1)
    a1 = _sc_aggregate(s1.reshape(NC * N_NODES, DH), srcz, dstr, zeros)
    s2 = _tc_act_matmul(a1, b1, W2)
    a2 = _sc_aggregate(s2.reshape(NC * N_NODES, DH), srcz, dstr, zeros)
    return _tc_bias(a2, b2)


# speedup vs baseline: 7.1073x; 1.4634x over previous
import functools

import jax
import jax.numpy as jnp
from jax import lax
from jax.experimental import pallas as pl
from jax.experimental.pallas import tpu as pltpu
from jax.experimental.pallas import tpu_sc as plsc

N_NODES = 10000
N_EDGES = 160000
D_FEAT = 256
EPSILON = 0.1
C = 10.0

NC = 2
NS = 16
DH = D_FEAT // NC
E_TILE = N_EDGES // NS
K = 80
CH = E_TILE // K
GRP = 25
NG = CH // GRP
GRP_PAD = 32
RCH = 80
NRC = N_NODES // RCH
NRC_TILE = (NRC + NS - 1) // NS


def _activation(x):
    mask = (x > EPSILON).astype(x.dtype)
    theta = (x - EPSILON) / (1.0 - EPSILON + 1e-8)
    theta = jnp.clip(theta, 0.0, 1.0)
    numerator = 1.0 + jnp.exp(jnp.asarray(-C, dtype=x.dtype))
    denominator = 1.0 + jnp.exp(-C * theta)
    return mask * (theta * numerator / denominator)


def _mm_body(x_ref, w_ref, o_ref):
    s = jnp.dot(x_ref[...], w_ref[...], preferred_element_type=jnp.float32)
    o_ref[0] = s[:, :DH]
    o_ref[1] = s[:, DH:]


def _tc_matmul(x, w):
    return pl.pallas_call(
        _mm_body,
        out_shape=jax.ShapeDtypeStruct((NC, N_NODES, DH), jnp.float32),
    )(x, w)


def _act_mm_body(a_ref, b_ref, w_ref, o_ref):
    a = jnp.concatenate([a_ref[0], a_ref[1]], axis=1) + b_ref[...]
    h = _activation(a)
    s = jnp.dot(h, w_ref[...], preferred_element_type=jnp.float32)
    o_ref[0] = s[:, :DH]
    o_ref[1] = s[:, DH:]


def _tc_act_matmul(agg, b, w):
    return pl.pallas_call(
        _act_mm_body,
        out_shape=jax.ShapeDtypeStruct((NC, N_NODES, DH), jnp.float32),
    )(agg, b, w)


def _bias_body(a_ref, b_ref, o_ref):
    o_ref[...] = jnp.concatenate([a_ref[0], a_ref[1]], axis=1) + b_ref[...]


def _tc_bias(agg, b):
    return pl.pallas_call(
        _bias_body,
        out_shape=jax.ShapeDtypeStruct((N_NODES, D_FEAT), jnp.float32),
    )(agg, b)


def _agg_body(support_ref, srcz_ref, dst_ref, zeros_ref, out_ref,
              src_v, dst_v, rows0, rows1, acc, sem0, sem1):
    c = lax.axis_index("c")
    s = lax.axis_index("s")

    def zero_body(i, carry):
        j = s * NRC_TILE + i

        @pl.when(j < NRC)
        def _():
            pltpu.sync_copy(zeros_ref, acc.at[pl.ds(j * RCH, RCH)])

        return carry

    lax.fori_loop(0, NRC_TILE, zero_body, 0)
    plsc.subcore_barrier()

    w = c * NS + s

    def group(g, carry):
        pltpu.sync_copy(srcz_ref.at[w, pl.ds(g * GRP_PAD, GRP_PAD)], src_v)
        pltpu.sync_copy(dst_ref.at[s, pl.ds(g * GRP_PAD, GRP_PAD)], dst_v)
        pltpu.async_copy(support_ref.at[src_v.at[0]], rows0, sem0)

        def body(t, inner):
            j0 = 2 * t
            pltpu.async_copy(support_ref.at[src_v.at[j0 + 1]], rows1, sem1)
            pltpu.make_async_copy(support_ref.at[src_v.at[j0]], rows0, sem0).wait()
            pltpu.sync_copy(rows0, acc.at[dst_v.at[j0]], add=True)
            pltpu.async_copy(support_ref.at[src_v.at[j0 + 2]], rows0, sem0)
            pltpu.make_async_copy(support_ref.at[src_v.at[j0 + 1]], rows1, sem1).wait()
            pltpu.sync_copy(rows1, acc.at[dst_v.at[j0 + 1]], add=True)
            return inner

        lax.fori_loop(0, (GRP - 1) // 2, body, 0)
        pltpu.make_async_copy(support_ref.at[src_v.at[GRP - 1]], rows0, sem0).wait()
        pltpu.sync_copy(rows0, acc.at[dst_v.at[GRP - 1]], add=True)
        return carry

    lax.fori_loop(0, NG, group, 0)
    plsc.subcore_barrier()

    def out_body(i, carry):
        j = s * NRC_TILE + i

        @pl.when(j < NRC)
        def _():
            pltpu.sync_copy(acc.at[pl.ds(j * RCH, RCH)],
                            out_ref.at[c, pl.ds(j * RCH, RCH)])

        return carry

    lax.fori_loop(0, NRC_TILE, out_body, 0)


_agg_kernel = functools.partial(
    pl.kernel,
    out_type=jax.ShapeDtypeStruct((NC, N_NODES, DH), jnp.float32),
    mesh=plsc.VectorSubcoreMesh(core_axis_name="c", subcore_axis_name="s"),
    scratch_types=[
        pltpu.VMEM((GRP_PAD, K), jnp.int32),
        pltpu.VMEM((GRP_PAD, K), jnp.int32),
        pltpu.VMEM((K, DH), jnp.float32),
        pltpu.VMEM((K, DH), jnp.float32),
        pltpu.VMEM_SHARED((N_NODES, DH), jnp.float32),
        pltpu.SemaphoreType.DMA,
        pltpu.SemaphoreType.DMA,
    ],
)(_agg_body)


def _sc_aggregate(support, srcz, dstr, zeros):
    return _agg_kernel(support, srcz, dstr, zeros)


def kernel(x, edge_index, W1, b1, W2, b2):
    src = edge_index[0].astype(jnp.int32)
    dst = edge_index[1].astype(jnp.int32)
    src_p = jnp.pad(src.reshape(NS, NG, GRP, K),
                    ((0, 0), (0, 0), (0, GRP_PAD - GRP), (0, 0)))
    dst_p = jnp.pad(dst.reshape(NS, NG, GRP, K),
                    ((0, 0), (0, 0), (0, GRP_PAD - GRP), (0, 0)))
    srcz = (src_p.reshape(1, NS, NG * GRP_PAD, K)
            + (jnp.arange(NC, dtype=jnp.int32) * N_NODES).reshape(NC, 1, 1, 1)
            ).reshape(NC * NS, NG * GRP_PAD, K)
    dstr = dst_p.reshape(NS, NG * GRP_PAD, K)
    zeros = jnp.zeros((RCH, DH), jnp.float32)

    s1 = _tc_matmul(x, W1)
    a1 = _sc_aggregate(s1.reshape(NC * N_NODES, DH), srcz, dstr, zeros)
    s2 = _tc_act_matmul(a1, b1, W2)
    a2 = _sc_aggregate(s2.reshape(NC * N_NODES, DH), srcz, dstr, zeros)
    return _tc_bias(a2, b2)


# speedup vs baseline: 7.5722x; 1.0654x over previous
import functools

import jax
import jax.numpy as jnp
from jax import lax
from jax.experimental import pallas as pl
from jax.experimental.pallas import tpu as pltpu
from jax.experimental.pallas import tpu_sc as plsc

N_NODES = 10000
N_EDGES = 160000
D_FEAT = 256
EPSILON = 0.1
C = 10.0

NC = 2
NS = 16
DH = D_FEAT // NC
E_TILE = N_EDGES // NS
K = 125
CH = E_TILE // K
GRP = 16
NG = CH // GRP
RCH = 80
NRC = N_NODES // RCH
NRC_TILE = (NRC + NS - 1) // NS


def _activation(x):
    mask = (x > EPSILON).astype(x.dtype)
    theta = (x - EPSILON) / (1.0 - EPSILON + 1e-8)
    theta = jnp.clip(theta, 0.0, 1.0)
    numerator = 1.0 + jnp.exp(jnp.asarray(-C, dtype=x.dtype))
    denominator = 1.0 + jnp.exp(-C * theta)
    return mask * (theta * numerator / denominator)


def _mm_body(x_ref, w_ref, o_ref):
    s = jnp.dot(x_ref[...], w_ref[...], preferred_element_type=jnp.float32)
    o_ref[0] = s[:, :DH]
    o_ref[1] = s[:, DH:]


def _tc_matmul(x, w):
    return pl.pallas_call(
        _mm_body,
        out_shape=jax.ShapeDtypeStruct((NC, N_NODES, DH), jnp.float32),
    )(x, w)


def _act_mm_body(a_ref, b_ref, w_ref, o_ref):
    a = jnp.concatenate([a_ref[0], a_ref[1]], axis=1) + b_ref[...]
    h = _activation(a)
    s = jnp.dot(h, w_ref[...], preferred_element_type=jnp.float32)
    o_ref[0] = s[:, :DH]
    o_ref[1] = s[:, DH:]


def _tc_act_matmul(agg, b, w):
    return pl.pallas_call(
        _act_mm_body,
        out_shape=jax.ShapeDtypeStruct((NC, N_NODES, DH), jnp.float32),
    )(agg, b, w)


def _bias_body(a_ref, b_ref, o_ref):
    o_ref[...] = jnp.concatenate([a_ref[0], a_ref[1]], axis=1) + b_ref[...]


def _tc_bias(agg, b):
    return pl.pallas_call(
        _bias_body,
        out_shape=jax.ShapeDtypeStruct((N_NODES, D_FEAT), jnp.float32),
    )(agg, b)


def _agg_body(support_ref, srcz_ref, dst_ref, zeros_ref, out_ref,
              src_v, dst_v, rows0, rows1, acc, sem0, sem1):
    c = lax.axis_index("c")
    s = lax.axis_index("s")

    def zero_body(i, carry):
        j = s * NRC_TILE + i

        @pl.when(j < NRC)
        def _():
            pltpu.sync_copy(zeros_ref, acc.at[pl.ds(j * RCH, RCH)])

        return carry

    lax.fori_loop(0, NRC_TILE, zero_body, 0)
    plsc.subcore_barrier()

    w = c * NS + s

    def group(g, carry):
        pltpu.sync_copy(srcz_ref.at[w, pl.ds(g * GRP, GRP)], src_v)
        pltpu.sync_copy(dst_ref.at[s, pl.ds(g * GRP, GRP)], dst_v)
        pltpu.async_copy(support_ref.at[src_v.at[0]], rows0, sem0)

        def body(t, inner):
            j0 = 2 * t
            pltpu.async_copy(support_ref.at[src_v.at[j0 + 1]], rows1, sem1)
            pltpu.make_async_copy(support_ref.at[src_v.at[j0]], rows0, sem0).wait()
            pltpu.sync_copy(rows0, acc.at[dst_v.at[j0]], add=True)

            @pl.when(j0 + 2 < GRP)
            def _():
                pltpu.async_copy(support_ref.at[src_v.at[j0 + 2]], rows0, sem0)

            pltpu.make_async_copy(support_ref.at[src_v.at[j0 + 1]], rows1, sem1).wait()
            pltpu.sync_copy(rows1, acc.at[dst_v.at[j0 + 1]], add=True)
            return inner

        lax.fori_loop(0, GRP // 2, body, 0)
        return carry

    lax.fori_loop(0, NG, group, 0)
    plsc.subcore_barrier()

    def out_body(i, carry):
        j = s * NRC_TILE + i

        @pl.when(j < NRC)
        def _():
            pltpu.sync_copy(acc.at[pl.ds(j * RCH, RCH)],
                            out_ref.at[c, pl.ds(j * RCH, RCH)])

        return carry

    lax.fori_loop(0, NRC_TILE, out_body, 0)


_agg_kernel = functools.partial(
    pl.kernel,
    out_type=jax.ShapeDtypeStruct((NC, N_NODES, DH), jnp.float32),
    mesh=plsc.VectorSubcoreMesh(core_axis_name="c", subcore_axis_name="s"),
    scratch_types=[
        pltpu.VMEM((GRP, K), jnp.int32),
        pltpu.VMEM((GRP, K), jnp.int32),
        pltpu.VMEM((K, DH), jnp.float32),
        pltpu.VMEM((K, DH), jnp.float32),
        pltpu.VMEM_SHARED((N_NODES, DH), jnp.float32),
        pltpu.SemaphoreType.DMA,
        pltpu.SemaphoreType.DMA,
    ],
)(_agg_body)


def _sc_aggregate(support, srcz, dstr, zeros):
    return _agg_kernel(support, srcz, dstr, zeros)


def kernel(x, edge_index, W1, b1, W2, b2):
    src = edge_index[0].astype(jnp.int32)
    dst = edge_index[1].astype(jnp.int32)
    srcz = (src.reshape(1, NS, CH, K)
            + (jnp.arange(NC, dtype=jnp.int32) * N_NODES).reshape(NC, 1, 1, 1)
            ).reshape(NC * NS, CH, K)
    dstr = dst.reshape(NS, CH, K)
    zeros = jnp.zeros((RCH, DH), jnp.float32)

    s1 = _tc_matmul(x, W1)
    a1 = _sc_aggregate(s1.reshape(NC * N_NODES, DH), srcz, dstr, zeros)
    s2 = _tc_act_matmul(a1, b1, W2)
    a2 = _sc_aggregate(s2.reshape(NC * N_NODES, DH), srcz, dstr, zeros)
    return _tc_bias(a2, b2)


# speedup vs baseline: 7.9184x; 1.0457x over previous
import functools

import jax
import jax.numpy as jnp
from jax import lax
from jax.experimental import pallas as pl
from jax.experimental.pallas import tpu as pltpu
from jax.experimental.pallas import tpu_sc as plsc

N_NODES = 10000
N_EDGES = 160000
D_FEAT = 256
EPSILON = 0.1
C = 10.0

NC = 2
NS = 16
DH = D_FEAT // NC
E_TILE = N_EDGES // NS
K = 80
CH = E_TILE // K
GRP = 25
NG = CH // GRP
GRP_PAD = 32
NB = 3
SIX = 6
RCH = 80
NRC = N_NODES // RCH
NRC_TILE = (NRC + NS - 1) // NS


def _activation(x):
    mask = (x > EPSILON).astype(x.dtype)
    theta = (x - EPSILON) / (1.0 - EPSILON + 1e-8)
    theta = jnp.clip(theta, 0.0, 1.0)
    numerator = 1.0 + jnp.exp(jnp.asarray(-C, dtype=x.dtype))
    denominator = 1.0 + jnp.exp(-C * theta)
    return mask * (theta * numerator / denominator)


def _mm_body(x_ref, w_ref, o_ref):
    s = jnp.dot(x_ref[...], w_ref[...], preferred_element_type=jnp.float32)
    o_ref[0] = s[:, :DH]
    o_ref[1] = s[:, DH:]


def _tc_matmul(x, w):
    return pl.pallas_call(
        _mm_body,
        out_shape=jax.ShapeDtypeStruct((NC, N_NODES, DH), jnp.float32),
    )(x, w)


def _act_mm_body(a_ref, b_ref, w_ref, o_ref):
    a = jnp.concatenate([a_ref[0], a_ref[1]], axis=1) + b_ref[...]
    h = _activation(a)
    s = jnp.dot(h, w_ref[...], preferred_element_type=jnp.float32)
    o_ref[0] = s[:, :DH]
    o_ref[1] = s[:, DH:]


def _tc_act_matmul(agg, b, w):
    return pl.pallas_call(
        _act_mm_body,
        out_shape=jax.ShapeDtypeStruct((NC, N_NODES, DH), jnp.float32),
    )(agg, b, w)


def _bias_body(a_ref, b_ref, o_ref):
    o_ref[...] = jnp.concatenate([a_ref[0], a_ref[1]], axis=1) + b_ref[...]


def _tc_bias(agg, b):
    return pl.pallas_call(
        _bias_body,
        out_shape=jax.ShapeDtypeStruct((N_NODES, D_FEAT), jnp.float32),
    )(agg, b)


def _agg_body(support_ref, srcz_ref, dst_ref, zeros_ref, out_ref,
              src_v, dst_v, rows0, rows1, rows2, acc,
              gsem0, gsem1, gsem2, ssem0, ssem1, ssem2):
    rows = (rows0, rows1, rows2)
    gsem = (gsem0, gsem1, gsem2)
    ssem = (ssem0, ssem1, ssem2)
    c = lax.axis_index("c")
    s = lax.axis_index("s")

    def zero_body(i, carry):
        j = s * NRC_TILE + i

        @pl.when(j < NRC)
        def _():
            pltpu.sync_copy(zeros_ref, acc.at[pl.ds(j * RCH, RCH)])

        return carry

    lax.fori_loop(0, NRC_TILE, zero_body, 0)
    plsc.subcore_barrier()

    w = c * NS + s

    def gather(j, b):
        pltpu.async_copy(support_ref.at[src_v.at[j]], rows[b], gsem[b])

    def gather_wait(j, b):
        pltpu.make_async_copy(support_ref.at[src_v.at[j]], rows[b], gsem[b]).wait()

    def scat(j, b):
        pltpu.async_copy(rows[b], acc.at[dst_v.at[j]], ssem[b], add=True)

    def scat_wait(j, b):
        pltpu.make_async_copy(rows[b], acc.at[dst_v.at[j]], ssem[b]).wait()

    def group(g, carry):
        pltpu.sync_copy(srcz_ref.at[w, pl.ds(g * GRP_PAD, GRP_PAD)], src_v)
        pltpu.sync_copy(dst_ref.at[s, pl.ds(g * GRP_PAD, GRP_PAD)], dst_v)
        gather(0, 0)
        gather(1, 1)

        def six(t, inner):
            for q in range(SIX):
                j = SIX * t + q
                b = q % NB
                gather_wait(j, b)
                scat(j, b)
                if q == 0:
                    @pl.when(j >= 1)
                    def _():
                        scat_wait(j - 1, (q - 1) % NB)
                else:
                    scat_wait(j - 1, (q - 1) % NB)

                @pl.when(j + 2 < GRP)
                def _():
                    gather(j + 2, (q + 2) % NB)
            return inner

        lax.fori_loop(0, (GRP - 1) // SIX, six, 0)
        gather_wait(GRP - 1, 0)
        scat(GRP - 1, 0)
        scat_wait(GRP - 2, 2)
        scat_wait(GRP - 1, 0)
        return carry

    lax.fori_loop(0, NG, group, 0)
    plsc.subcore_barrier()

    def out_body(i, carry):
        j = s * NRC_TILE + i

        @pl.when(j < NRC)
        def _():
            pltpu.sync_copy(acc.at[pl.ds(j * RCH, RCH)],
                            out_ref.at[c, pl.ds(j * RCH, RCH)])

        return carry

    lax.fori_loop(0, NRC_TILE, out_body, 0)


_agg_kernel = functools.partial(
    pl.kernel,
    out_type=jax.ShapeDtypeStruct((NC, N_NODES, DH), jnp.float32),
    mesh=plsc.VectorSubcoreMesh(core_axis_name="c", subcore_axis_name="s"),
    scratch_types=[
        pltpu.VMEM((GRP_PAD, K), jnp.int32),
        pltpu.VMEM((GRP_PAD, K), jnp.int32),
        pltpu.VMEM((K, DH), jnp.float32),
        pltpu.VMEM((K, DH), jnp.float32),
        pltpu.VMEM((K, DH), jnp.float32),
        pltpu.VMEM_SHARED((N_NODES, DH), jnp.float32),
        pltpu.SemaphoreType.DMA,
        pltpu.SemaphoreType.DMA,
        pltpu.SemaphoreType.DMA,
        pltpu.SemaphoreType.DMA,
        pltpu.SemaphoreType.DMA,
        pltpu.SemaphoreType.DMA,
    ],
)(_agg_body)


def _sc_aggregate(support, srcz, dstr, zeros):
    return _agg_kernel(support, srcz, dstr, zeros)


def kernel(x, edge_index, W1, b1, W2, b2):
    src = edge_index[0].astype(jnp.int32)
    dst = edge_index[1].astype(jnp.int32)
    src_p = jnp.pad(src.reshape(NS, NG, GRP, K),
                    ((0, 0), (0, 0), (0, GRP_PAD - GRP), (0, 0)))
    dst_p = jnp.pad(dst.reshape(NS, NG, GRP, K),
                    ((0, 0), (0, 0), (0, GRP_PAD - GRP), (0, 0)))
    srcz = (src_p.reshape(1, NS, NG * GRP_PAD, K)
            + (jnp.arange(NC, dtype=jnp.int32) * N_NODES).reshape(NC, 1, 1, 1)
            ).reshape(NC * NS, NG * GRP_PAD, K)
    dstr = dst_p.reshape(NS, NG * GRP_PAD, K)
    zeros = jnp.zeros((RCH, DH), jnp.float32)

    s1 = _tc_matmul(x, W1)
    a1 = _sc_aggregate(s1.reshape(NC * N_NODES, DH), srcz, dstr, zeros)
    s2 = _tc_act_matmul(a1, b1, W2)
    a2 = _sc_aggregate(s2.reshape(NC * N_NODES, DH), srcz, dstr, zeros)
    return _tc_bias(a2, b2)


# speedup vs baseline: 8.1394x; 1.0279x over previous
import functools

import jax
import jax.numpy as jnp
from jax import lax
from jax.experimental import pallas as pl
from jax.experimental.pallas import tpu as pltpu
from jax.experimental.pallas import tpu_sc as plsc

N_NODES = 10000
N_EDGES = 160000
D_FEAT = 256
EPSILON = 0.1
C = 10.0

NC = 2
NS = 16
DH = D_FEAT // NC
E_TILE = N_EDGES // NS
K = 80
CH = E_TILE // K
GRP = 25
NG = CH // GRP
GRP_PAD = 32
NB = 3
SIX = 6
RCH = 80
NRC = N_NODES // RCH
NRC_TILE = (NRC + NS - 1) // NS


def _activation(x):
    mask = (x > EPSILON).astype(x.dtype)
    theta = (x - EPSILON) / (1.0 - EPSILON + 1e-8)
    theta = jnp.clip(theta, 0.0, 1.0)
    numerator = 1.0 + jnp.exp(jnp.asarray(-C, dtype=x.dtype))
    denominator = 1.0 + jnp.exp(-C * theta)
    return mask * (theta * numerator / denominator)


def _mm_act_body(a_ref, w_ref, b_ref, o_ref):
    a = jnp.concatenate([a_ref[0], a_ref[1]], axis=1)
    t = jnp.dot(a, w_ref[...], preferred_element_type=jnp.float32) + b_ref[...]
    o_ref[...] = _activation(t)


def _tc_mm_act(agg, w, b):
    return pl.pallas_call(
        _mm_act_body,
        out_shape=jax.ShapeDtypeStruct((N_NODES, D_FEAT), jnp.float32),
    )(agg, w, b)


def _mm_bias_body(a_ref, w_ref, b_ref, o_ref):
    a = jnp.concatenate([a_ref[0], a_ref[1]], axis=1)
    o_ref[...] = jnp.dot(a, w_ref[...],
                         preferred_element_type=jnp.float32) + b_ref[...]


def _tc_mm_bias(agg, w, b):
    return pl.pallas_call(
        _mm_bias_body,
        out_shape=jax.ShapeDtypeStruct((N_NODES, D_FEAT), jnp.float32),
    )(agg, w, b)


def _agg_body(table_ref, src_ref, dst_ref, zeros_ref, out_ref,
              src_v, dst_v, rows0, rows1, rows2, acc,
              gsem0, gsem1, gsem2, ssem0, ssem1, ssem2):
    rows = (rows0, rows1, rows2)
    gsem = (gsem0, gsem1, gsem2)
    ssem = (ssem0, ssem1, ssem2)
    c = lax.axis_index("c")
    s = lax.axis_index("s")

    def zero_body(i, carry):
        j = s * NRC_TILE + i

        @pl.when(j < NRC)
        def _():
            pltpu.sync_copy(zeros_ref, acc.at[pl.ds(j * RCH, RCH)])

        return carry

    lax.fori_loop(0, NRC_TILE, zero_body, 0)
    plsc.subcore_barrier()

    col = c * DH

    def gather(j, b):
        pltpu.async_copy(table_ref.at[src_v.at[j], pl.ds(col, DH)],
                         rows[b], gsem[b])

    def gather_wait(j, b):
        pltpu.make_async_copy(table_ref.at[src_v.at[j], pl.ds(col, DH)],
                              rows[b], gsem[b]).wait()

    def scat(j, b):
        pltpu.async_copy(rows[b], acc.at[dst_v.at[j]], ssem[b], add=True)

    def scat_wait(j, b):
        pltpu.make_async_copy(rows[b], acc.at[dst_v.at[j]], ssem[b]).wait()

    def group(g, carry):
        pltpu.sync_copy(src_ref.at[s, pl.ds(g * GRP_PAD, GRP_PAD)], src_v)
        pltpu.sync_copy(dst_ref.at[s, pl.ds(g * GRP_PAD, GRP_PAD)], dst_v)
        gather(0, 0)
        gather(1, 1)

        def six(t, inner):
            for q in range(SIX):
                j = SIX * t + q
                b = q % NB
                gather_wait(j, b)
                scat(j, b)
                if q == 0:
                    @pl.when(j >= 1)
                    def _():
                        scat_wait(j - 1, (q - 1) % NB)
                else:
                    scat_wait(j - 1, (q - 1) % NB)

                @pl.when(j + 2 < GRP)
                def _():
                    gather(j + 2, (q + 2) % NB)
            return inner

        lax.fori_loop(0, (GRP - 1) // SIX, six, 0)
        gather_wait(GRP - 1, 0)
        scat(GRP - 1, 0)
        scat_wait(GRP - 2, 2)
        scat_wait(GRP - 1, 0)
        return carry

    lax.fori_loop(0, NG, group, 0)
    plsc.subcore_barrier()

    def out_body(i, carry):
        j = s * NRC_TILE + i

        @pl.when(j < NRC)
        def _():
            pltpu.sync_copy(acc.at[pl.ds(j * RCH, RCH)],
                            out_ref.at[c, pl.ds(j * RCH, RCH)])

        return carry

    lax.fori_loop(0, NRC_TILE, out_body, 0)


_agg_kernel = functools.partial(
    pl.kernel,
    out_type=jax.ShapeDtypeStruct((NC, N_NODES, DH), jnp.float32),
    mesh=plsc.VectorSubcoreMesh(core_axis_name="c", subcore_axis_name="s"),
    scratch_types=[
        pltpu.VMEM((GRP_PAD, K), jnp.int32),
        pltpu.VMEM((GRP_PAD, K), jnp.int32),
        pltpu.VMEM((K, DH), jnp.float32),
        pltpu.VMEM((K, DH), jnp.float32),
        pltpu.VMEM((K, DH), jnp.float32),
        pltpu.VMEM_SHARED((N_NODES, DH), jnp.float32),
        pltpu.SemaphoreType.DMA,
        pltpu.SemaphoreType.DMA,
        pltpu.SemaphoreType.DMA,
        pltpu.SemaphoreType.DMA,
        pltpu.SemaphoreType.DMA,
        pltpu.SemaphoreType.DMA,
    ],
)(_agg_body)


def _sc_aggregate(table, srcr, dstr, zeros):
    return _agg_kernel(table, srcr, dstr, zeros)


def kernel(x, edge_index, W1, b1, W2, b2):
    src = edge_index[0].astype(jnp.int32)
    dst = edge_index[1].astype(jnp.int32)
    srcr = jnp.pad(src.reshape(NS, NG, GRP, K),
                   ((0, 0), (0, 0), (0, GRP_PAD - GRP), (0, 0))
                   ).reshape(NS, NG * GRP_PAD, K)
    dstr = jnp.pad(dst.reshape(NS, NG, GRP, K),
                   ((0, 0), (0, 0), (0, GRP_PAD - GRP), (0, 0))
                   ).reshape(NS, NG * GRP_PAD, K)
    zeros = jnp.zeros((RCH, DH), jnp.float32)

    a1 = _sc_aggregate(x, srcr, dstr, zeros)
    h = _tc_mm_act(a1, W1, b1)
    a2 = _sc_aggregate(h, srcr, dstr, zeros)
    return _tc_mm_bias(a2, W2, b2)


# speedup vs baseline: 8.2383x; 1.0122x over previous
import functools

import jax
import jax.numpy as jnp
from jax import lax
from jax.experimental import pallas as pl
from jax.experimental.pallas import tpu as pltpu
from jax.experimental.pallas import tpu_sc as plsc

N_NODES = 10000
N_EDGES = 160000
D_FEAT = 256
EPSILON = 0.1
C = 10.0

NC = 2
NS = 16
DH = D_FEAT // NC
E_TILE = N_EDGES // NS
K = 80
CH = E_TILE // K
GRP = 25
NG = CH // GRP
GRP_PAD = 32
NB = 3
SIX = 3
RCH = 80
NRC = N_NODES // RCH
NRC_TILE = (NRC + NS - 1) // NS


def _activation(x):
    mask = (x > EPSILON).astype(x.dtype)
    theta = (x - EPSILON) / (1.0 - EPSILON + 1e-8)
    theta = jnp.clip(theta, 0.0, 1.0)
    numerator = 1.0 + jnp.exp(jnp.asarray(-C, dtype=x.dtype))
    denominator = 1.0 + jnp.exp(-C * theta)
    return mask * (theta * numerator / denominator)


def _mm_act_body(a_ref, w_ref, b_ref, o_ref):
    a = jnp.concatenate([a_ref[0], a_ref[1]], axis=1)
    t = jnp.dot(a, w_ref[...], preferred_element_type=jnp.float32) + b_ref[...]
    o_ref[...] = _activation(t)


def _tc_mm_act(agg, w, b):
    return pl.pallas_call(
        _mm_act_body,
        out_shape=jax.ShapeDtypeStruct((N_NODES, D_FEAT), jnp.float32),
    )(agg, w, b)


def _mm_bias_body(a_ref, w_ref, b_ref, o_ref):
    a = jnp.concatenate([a_ref[0], a_ref[1]], axis=1)
    o_ref[...] = jnp.dot(a, w_ref[...],
                         preferred_element_type=jnp.float32) + b_ref[...]


def _tc_mm_bias(agg, w, b):
    return pl.pallas_call(
        _mm_bias_body,
        out_shape=jax.ShapeDtypeStruct((N_NODES, D_FEAT), jnp.float32),
    )(agg, w, b)


def _agg_body(table_ref, src_ref, dst_ref, zeros_ref, out_ref,
              src_v0, dst_v0, src_v1, dst_v1, rows0, rows1, rows2, acc,
              gsem0, gsem1, gsem2, ssem0, ssem1, ssem2, isem):
    rows = (rows0, rows1, rows2)
    gsem = (gsem0, gsem1, gsem2)
    ssem = (ssem0, ssem1, ssem2)
    sv = (src_v0, src_v1)
    dv = (dst_v0, dst_v1)
    c = lax.axis_index("c")
    s = lax.axis_index("s")
    col = c * DH


    def gather(ib, j, b):
        pltpu.async_copy(table_ref.at[sv[ib].at[j], pl.ds(col, DH)],
                         rows[b], gsem[b])

    def gather_wait(ib, j, b):
        pltpu.make_async_copy(table_ref.at[sv[ib].at[j], pl.ds(col, DH)],
                              rows[b], gsem[b]).wait()

    def scat(ib, j, b):
        pltpu.async_copy(rows[b], acc.at[dv[ib].at[j]], ssem[b], add=True)

    def scat_wait(ib, j, b):
        pltpu.make_async_copy(rows[b], acc.at[dv[ib].at[j]], ssem[b]).wait()

    def refill_async(g, ib):
        pltpu.async_copy(src_ref.at[s, pl.ds(g * GRP_PAD, GRP_PAD)],
                         sv[ib], isem)
        pltpu.async_copy(dst_ref.at[s, pl.ds(g * GRP_PAD, GRP_PAD)],
                         dv[ib], isem)

    def refill_wait(g, ib):
        pltpu.make_async_copy(src_ref.at[s, pl.ds(g * GRP_PAD, GRP_PAD)],
                              sv[ib], isem).wait()
        pltpu.make_async_copy(dst_ref.at[s, pl.ds(g * GRP_PAD, GRP_PAD)],
                              dv[ib], isem).wait()

    pltpu.sync_copy(src_ref.at[s, pl.ds(0, GRP_PAD)], src_v0)
    pltpu.sync_copy(dst_ref.at[s, pl.ds(0, GRP_PAD)], dst_v0)
    gather(0, 0, 0)
    gather(0, 1, 1)

    def zero_body(i, carry):
        j = s * NRC_TILE + i

        @pl.when(j < NRC)
        def _():
            pltpu.sync_copy(zeros_ref, acc.at[pl.ds(j * RCH, RCH)])

        return carry

    lax.fori_loop(0, NRC_TILE, zero_body, 0)
    plsc.subcore_barrier()

    for g in range(NG):
        ib = g % 2
        if g + 1 < NG:
            refill_async(g + 1, 1 - ib)

        def six(t, inner, ib=ib):
            for q in range(SIX):
                j = SIX * t + q
                b = q % NB
                gather_wait(ib, j, b)
                scat(ib, j, b)
                if q == 0:
                    @pl.when(j >= 1)
                    def _():
                        scat_wait(ib, j - 1, (q - 1) % NB)
                else:
                    scat_wait(ib, j - 1, (q - 1) % NB)

                @pl.when(j + 2 < GRP)
                def _():
                    gather(ib, j + 2, (q + 2) % NB)
            return inner

        lax.fori_loop(0, (GRP - 1) // SIX, six, 0)
        gather_wait(ib, GRP - 1, 0)
        scat(ib, GRP - 1, 0)
        scat_wait(ib, GRP - 2, 2)
        scat_wait(ib, GRP - 1, 0)
        if g + 1 < NG:
            refill_wait(g + 1, 1 - ib)
            gather(1 - ib, 0, 0)
            gather(1 - ib, 1, 1)

    plsc.subcore_barrier()

    def out_body(i, carry):
        j = s * NRC_TILE + i

        @pl.when(j < NRC)
        def _():
            pltpu.sync_copy(acc.at[pl.ds(j * RCH, RCH)],
                            out_ref.at[c, pl.ds(j * RCH, RCH)])

        return carry

    lax.fori_loop(0, NRC_TILE, out_body, 0)


_agg_kernel = functools.partial(
    pl.kernel,
    out_type=jax.ShapeDtypeStruct((NC, N_NODES, DH), jnp.float32),
    mesh=plsc.VectorSubcoreMesh(core_axis_name="c", subcore_axis_name="s"),
    scratch_types=[
        pltpu.VMEM((GRP_PAD, K), jnp.int32),
        pltpu.VMEM((GRP_PAD, K), jnp.int32),
        pltpu.VMEM((GRP_PAD, K), jnp.int32),
        pltpu.VMEM((GRP_PAD, K), jnp.int32),
        pltpu.VMEM((K, DH), jnp.float32),
        pltpu.VMEM((K, DH), jnp.float32),
        pltpu.VMEM((K, DH), jnp.float32),
        pltpu.VMEM_SHARED((N_NODES, DH), jnp.float32),
        pltpu.SemaphoreType.DMA,
        pltpu.SemaphoreType.DMA,
        pltpu.SemaphoreType.DMA,
        pltpu.SemaphoreType.DMA,
        pltpu.SemaphoreType.DMA,
        pltpu.SemaphoreType.DMA,
        pltpu.SemaphoreType.DMA,
    ],
)(_agg_body)


def _sc_aggregate(table, srcr, dstr, zeros):
    return _agg_kernel(table, srcr, dstr, zeros)


def kernel(x, edge_index, W1, b1, W2, b2):
    src = edge_index[0].astype(jnp.int32)
    dst = edge_index[1].astype(jnp.int32)
    srcr = jnp.pad(src.reshape(NS, NG, GRP, K),
                   ((0, 0), (0, 0), (0, GRP_PAD - GRP), (0, 0))
                   ).reshape(NS, NG * GRP_PAD, K)
    dstr = jnp.pad(dst.reshape(NS, NG, GRP, K),
                   ((0, 0), (0, 0), (0, GRP_PAD - GRP), (0, 0))
                   ).reshape(NS, NG * GRP_PAD, K)
    zeros = jnp.zeros((RCH, DH), jnp.float32)

    a1 = _sc_aggregate(x, srcr, dstr, zeros)
    h = _tc_mm_act(a1, W1, b1)
    a2 = _sc_aggregate(h, srcr, dstr, zeros)
    return _tc_mm_bias(a2, W2, b2)
